# baseline (speedup 1.0000x reference)
import functools

import jax
import jax.numpy as jnp
from jax import lax
from jax.experimental import pallas as pl
from jax.experimental.pallas import tpu as pltpu
from jax.experimental.pallas import tpu_sc as plsc

B = 1024
D = 256
C = 100
M = 100000
BM = 2000
NBLK = M // BM

_NEG_INF = float("-inf")


def _main_body(q_ref, qn_ref_in, k_ref, kn_ref_in, conf_ref, idx_ref,
               qn_ref, max_ref, arg_ref):
    step = pl.program_id(0)

    @pl.when(step == 0)
    def _init():
        qn_ref[...] = q_ref[...] / jnp.maximum(qn_ref_in[...], 1e-12)
        max_ref[...] = jnp.full((B, 1), _NEG_INF, jnp.float32)
        arg_ref[...] = jnp.zeros((B, 1), jnp.int32)

    kn = k_ref[...] / jnp.maximum(kn_ref_in[...], 1e-12)
    s = lax.dot_general(qn_ref[...], kn, (((1,), (1,)), ((), ())),
                        preferred_element_type=jnp.float32)
    bmax = jnp.max(s, axis=1, keepdims=True)
    ii = lax.broadcasted_iota(jnp.int32, (B, BM), 1)
    barg = jnp.min(jnp.where(s == bmax, ii, BM), axis=1, keepdims=True)

    run_max = max_ref[...]
    better = bmax > run_max
    max_ref[...] = jnp.where(better, bmax, run_max)
    arg_ref[...] = jnp.where(better, barg + step * BM, arg_ref[...])

    @pl.when(step == NBLK - 1)
    def _fin():
        conf_ref[...] = max_ref[...]
        idx_ref[...] = arg_ref[...]


_main_call = pl.pallas_call(
    _main_body,
    grid=(NBLK,),
    in_specs=[
        pl.BlockSpec((B, D), lambda i: (0, 0)),
        pl.BlockSpec((B, 1), lambda i: (0, 0)),
        pl.BlockSpec((BM, D), lambda i: (i, 0)),
        pl.BlockSpec((BM, 1), lambda i: (i, 0)),
    ],
    out_specs=[
        pl.BlockSpec((B, 1), lambda i: (0, 0)),
        pl.BlockSpec((B, 1), lambda i: (0, 0)),
    ],
    out_shape=[
        jax.ShapeDtypeStruct((B, 1), jnp.float32),
        jax.ShapeDtypeStruct((B, 1), jnp.int32),
    ],
    scratch_shapes=[
        pltpu.VMEM((B, D), jnp.float32),
        pltpu.VMEM((B, 1), jnp.float32),
        pltpu.VMEM((B, 1), jnp.int32),
    ],
    compiler_params=pltpu.CompilerParams(
        dimension_semantics=("arbitrary",),
    ),
)


def _cls_body(a_ref, b_ref, idx_ref, out_ref):
    x = jnp.concatenate([a_ref[...], b_ref[...]], axis=1)
    off = (idx_ref[...] * C) & 127
    cc = lax.broadcasted_iota(jnp.int32, (B, 256), 1)
    inw = (cc >= off) & (cc < off + C)
    xm = jnp.where(inw, x, _NEG_INF)
    m = jnp.max(xm, axis=1, keepdims=True)
    out_ref[...] = jnp.min(jnp.where(xm == m, cc - off, 256),
                           axis=1, keepdims=True)


_cls_call = pl.pallas_call(
    _cls_body,
    out_shape=jax.ShapeDtypeStruct((B, 1), jnp.int32),
)


_NC = 2
_NS = 16
_NW = _NC * _NS
_BPW = B // _NW

_VROWS = (M * C) // 128


@functools.cache
def _make_sc_retrieve():
    mesh = plsc.VectorSubcoreMesh(core_axis_name="c", subcore_axis_name="s")

    @functools.partial(
        pl.kernel,
        mesh=mesh,
        out_type=[
            jax.ShapeDtypeStruct((B, 128), jnp.float32),
            jax.ShapeDtypeStruct((B, 128), jnp.float32),
        ],
        scratch_types=[
            pltpu.VMEM((_BPW,), jnp.int32),
            pltpu.VMEM((2 * _BPW,), jnp.int32),
            pltpu.VMEM((2 * _BPW, 128), jnp.float32),
            pltpu.SemaphoreType.DMA,
        ],
    )
    def _sc_retrieve(table_hbm, idx_hbm, outa_hbm, outb_hbm, idx_v, ind2_v,
                     rows_v, sem):
        wid = lax.axis_index("s") * _NC + lax.axis_index("c")
        base = wid * _BPW
        pltpu.sync_copy(idx_hbm.at[pl.ds(base, _BPW)], idx_v)
        for c in range(_BPW // 16):
            r = idx_v[pl.ds(16 * c, 16)]
            g = lax.shift_right_logical(r * C, 7)
            ind2_v[pl.ds(16 * c, 16)] = g
            ind2_v[pl.ds(_BPW + 16 * c, 16)] = jnp.minimum(g + 1, _VROWS - 1)
        pltpu.async_copy(table_hbm.at[ind2_v], rows_v, sem).wait()
        pltpu.sync_copy(rows_v.at[pl.ds(0, _BPW)],
                        outa_hbm.at[pl.ds(base, _BPW)])
        pltpu.sync_copy(rows_v.at[pl.ds(_BPW, _BPW)],
                        outb_hbm.at[pl.ds(base, _BPW)])

    return _sc_retrieve


def kernel(query_features, memory_keys, memory_values):
    qnorm = jnp.linalg.norm(query_features, ord=2, axis=1, keepdims=True)
    knorm = jnp.linalg.norm(memory_keys, ord=2, axis=1, keepdims=True)
    conf, idx = _main_call(query_features, qnorm, memory_keys, knorm)
    table = memory_values.reshape(_VROWS, 128)
    ga, gb = _make_sc_retrieve()(table, idx.reshape(B))
    classes = _cls_call(ga, gb, idx)
    return classes.reshape(B), conf.reshape(B)


# speedup vs baseline: 1.1385x; 1.1385x over previous
import functools

import jax
import jax.numpy as jnp
from jax import lax
from jax.experimental import pallas as pl
from jax.experimental.pallas import tpu as pltpu
from jax.experimental.pallas import tpu_sc as plsc

B = 1024
D = 256
C = 100
M = 100000
BM = 2000
NBLK = M // BM

_NEG_INF = float("-inf")


def _main_body(q_ref, qn_ref_in, k_ref, kn_ref_in, conf_ref, idx_ref,
               qn_ref, max_ref, arg_ref):
    step = pl.program_id(0)

    @pl.when(step == 0)
    def _init():
        qn_ref[...] = q_ref[...] / jnp.maximum(qn_ref_in[...], 1e-12)
        max_ref[...] = jnp.full((B, 1), _NEG_INF, jnp.float32)
        arg_ref[...] = jnp.zeros((B, 1), jnp.int32)

    kn = k_ref[...] / jnp.maximum(kn_ref_in[...], 1e-12)
    s = lax.dot_general(qn_ref[...], kn, (((1,), (1,)), ((), ())),
                        preferred_element_type=jnp.float32)
    bmax = jnp.max(s, axis=1, keepdims=True)
    ii = lax.broadcasted_iota(jnp.int32, (B, BM), 1)
    barg = jnp.min(jnp.where(s == bmax, ii, BM), axis=1, keepdims=True)

    run_max = max_ref[...]
    better = bmax > run_max
    max_ref[...] = jnp.where(better, bmax, run_max)
    arg_ref[...] = jnp.where(better, barg + step * BM, arg_ref[...])

    @pl.when(step == NBLK - 1)
    def _fin():
        conf_ref[...] = max_ref[...]
        idx_ref[...] = arg_ref[...]


_main_call = pl.pallas_call(
    _main_body,
    grid=(NBLK,),
    in_specs=[
        pl.BlockSpec((B, D), lambda i: (0, 0)),
        pl.BlockSpec((B, 1), lambda i: (0, 0)),
        pl.BlockSpec((BM, D), lambda i: (i, 0)),
        pl.BlockSpec((BM, 1), lambda i: (i, 0)),
    ],
    out_specs=[
        pl.BlockSpec((B, 1), lambda i: (0, 0)),
        pl.BlockSpec((B, 1), lambda i: (0, 0)),
    ],
    out_shape=[
        jax.ShapeDtypeStruct((B, 1), jnp.float32),
        jax.ShapeDtypeStruct((B, 1), jnp.int32),
    ],
    scratch_shapes=[
        pltpu.VMEM((B, D), jnp.float32),
        pltpu.VMEM((B, 1), jnp.float32),
        pltpu.VMEM((B, 1), jnp.int32),
    ],
    compiler_params=pltpu.CompilerParams(
        dimension_semantics=("arbitrary",),
    ),
)


def _cls_body(g_ref, out_ref):
    g = g_ref[...]
    m = jnp.max(g, axis=1, keepdims=True)
    ii = lax.broadcasted_iota(jnp.int32, (B, C), 1)
    out_ref[...] = jnp.min(jnp.where(g == m, ii, C), axis=1, keepdims=True)


_cls_call = pl.pallas_call(
    _cls_body,
    out_shape=jax.ShapeDtypeStruct((B, 1), jnp.int32),
)


_NC = 2
_NS = 16
_NW = _NC * _NS
_BPW = B // _NW


@functools.cache
def _make_sc_retrieve():
    mesh = plsc.VectorSubcoreMesh(core_axis_name="c", subcore_axis_name="s")

    @functools.partial(
        pl.kernel,
        mesh=mesh,
        out_type=jax.ShapeDtypeStruct((B, C), jnp.float32),
        scratch_types=[
            pltpu.VMEM((_BPW,), jnp.int32),
            pltpu.VMEM((_BPW, C), jnp.float32),
            pltpu.SemaphoreType.DMA,
        ],
    )
    def _sc_retrieve(table_hbm, idx_hbm, out_hbm, idx_v, rows_v, sem):
        wid = lax.axis_index("s") * _NC + lax.axis_index("c")
        base = wid * _BPW
        pltpu.sync_copy(idx_hbm.at[pl.ds(base, _BPW)], idx_v)
        chunks = [idx_v[pl.ds(16 * c, 16)] for c in range(_BPW // 16)]
        copies = []
        for i in range(_BPW):
            r = chunks[i // 16][i % 16]
            copies.append(
                pltpu.async_copy(table_hbm.at[r], rows_v.at[i], sem))
        for cp in copies:
            cp.wait()
        pltpu.sync_copy(rows_v, out_hbm.at[pl.ds(base, _BPW)])

    return _sc_retrieve


def kernel(query_features, memory_keys, memory_values):
    qnorm = jnp.linalg.norm(query_features, ord=2, axis=1, keepdims=True)
    knorm = jnp.linalg.norm(memory_keys, ord=2, axis=1, keepdims=True)
    conf, idx = _main_call(query_features, qnorm, memory_keys, knorm)
    gathered = _make_sc_retrieve()(memory_values, idx.reshape(B))
    classes = _cls_call(gathered)
    return classes.reshape(B), conf.reshape(B)


# speedup vs baseline: 1.2633x; 1.1096x over previous
import functools

import jax
import jax.numpy as jnp
from jax import lax
from jax.experimental import pallas as pl
from jax.experimental.pallas import tpu as pltpu
from jax.experimental.pallas import tpu_sc as plsc

B = 1024
D = 256
C = 100
M = 100000
BM = 5000
NBLK = M // BM

_NEG_INF = float("-inf")


def _main_body(q_ref, qn_ref_in, k_ref, kn_ref_in, conf_ref, idx_ref,
               qn_ref, max_ref, arg_ref):
    step = pl.program_id(0)

    @pl.when(step == 0)
    def _init():
        qn_ref[...] = q_ref[...] / jnp.maximum(qn_ref_in[...], 1e-12)
        max_ref[...] = jnp.full((B, 1), _NEG_INF, jnp.float32)
        arg_ref[...] = jnp.zeros((B, 1), jnp.int32)

    kn = k_ref[...] / jnp.maximum(kn_ref_in[...], 1e-12)
    s = lax.dot_general(qn_ref[...], kn, (((1,), (1,)), ((), ())),
                        preferred_element_type=jnp.float32)
    bmax = jnp.max(s, axis=1, keepdims=True)
    fdesc = (BM - lax.broadcasted_iota(jnp.int32, (1, BM), 1)).astype(
        jnp.float32)
    fmax = jnp.max(jnp.where(s == bmax, fdesc, 0.0), axis=1, keepdims=True)
    barg = BM - fmax.astype(jnp.int32)

    run_max = max_ref[...]
    better = bmax > run_max
    max_ref[...] = jnp.where(better, bmax, run_max)
    arg_ref[...] = jnp.where(better, barg + step * BM, arg_ref[...])

    @pl.when(step == NBLK - 1)
    def _fin():
        conf_ref[...] = max_ref[...]
        idx_ref[...] = arg_ref[...]


_main_call = pl.pallas_call(
    _main_body,
    grid=(NBLK,),
    in_specs=[
        pl.BlockSpec((B, D), lambda i: (0, 0)),
        pl.BlockSpec((B, 1), lambda i: (0, 0)),
        pl.BlockSpec((BM, D), lambda i: (i, 0)),
        pl.BlockSpec((BM, 1), lambda i: (i, 0)),
    ],
    out_specs=[
        pl.BlockSpec((B, 1), lambda i: (0, 0)),
        pl.BlockSpec((B, 1), lambda i: (0, 0)),
    ],
    out_shape=[
        jax.ShapeDtypeStruct((B, 1), jnp.float32),
        jax.ShapeDtypeStruct((B, 1), jnp.int32),
    ],
    scratch_shapes=[
        pltpu.VMEM((B, D), jnp.float32),
        pltpu.VMEM((B, 1), jnp.float32),
        pltpu.VMEM((B, 1), jnp.int32),
    ],
    compiler_params=pltpu.CompilerParams(
        dimension_semantics=("arbitrary",),
    ),
)


def _cls_body(g_ref, out_ref):
    g = g_ref[...]
    m = jnp.max(g, axis=1, keepdims=True)
    ii = lax.broadcasted_iota(jnp.int32, (B, C), 1)
    out_ref[...] = jnp.min(jnp.where(g == m, ii, C), axis=1, keepdims=True)


_cls_call = pl.pallas_call(
    _cls_body,
    out_shape=jax.ShapeDtypeStruct((B, 1), jnp.int32),
)


_NC = 2
_NS = 16
_NW = _NC * _NS
_BPW = B // _NW


@functools.cache
def _make_sc_retrieve():
    mesh = plsc.VectorSubcoreMesh(core_axis_name="c", subcore_axis_name="s")

    @functools.partial(
        pl.kernel,
        mesh=mesh,
        out_type=jax.ShapeDtypeStruct((B, C), jnp.float32),
        scratch_types=[
            pltpu.VMEM((_BPW,), jnp.int32),
            pltpu.VMEM((_BPW, C), jnp.float32),
            pltpu.SemaphoreType.DMA,
        ],
    )
    def _sc_retrieve(table_hbm, idx_hbm, out_hbm, idx_v, rows_v, sem):
        wid = lax.axis_index("s") * _NC + lax.axis_index("c")
        base = wid * _BPW
        pltpu.sync_copy(idx_hbm.at[pl.ds(base, _BPW)], idx_v)
        chunks = [idx_v[pl.ds(16 * c, 16)] for c in range(_BPW // 16)]
        copies = []
        for i in range(_BPW):
            r = chunks[i // 16][i % 16]
            copies.append(
                pltpu.async_copy(table_hbm.at[r], rows_v.at[i], sem))
        for cp in copies:
            cp.wait()
        pltpu.sync_copy(rows_v, out_hbm.at[pl.ds(base, _BPW)])

    return _sc_retrieve


def kernel(query_features, memory_keys, memory_values):
    qnorm = jnp.linalg.norm(query_features, ord=2, axis=1, keepdims=True)
    knorm = jnp.linalg.norm(memory_keys, ord=2, axis=1, keepdims=True)
    conf, idx = _main_call(query_features, qnorm, memory_keys, knorm)
    gathered = _make_sc_retrieve()(memory_values, idx.reshape(B))
    classes = _cls_call(gathered)
    return classes.reshape(B), conf.reshape(B)
